# Initial kernel scaffold; baseline (speedup 1.0000x reference)
import jax
import jax.numpy as jnp
from jax.experimental import pallas as pl


def kernel(pc1, pc2, est_flow):
    raise NotImplementedError("write your pallas kernel here")


# speedup vs baseline: 2.8348x; 2.8348x over previous
import functools

import jax
import jax.numpy as jnp
from jax.experimental import pallas as pl
from jax.experimental.pallas import tpu as pltpu

_N = 4096
_TI = 256


def _chamfer_body(w_ref, p_ref, cham_ref, idx_ref, colmin_ref, loss_ref,
                  rowsum_ref):
    step = pl.program_id(0)
    nsteps = pl.num_programs(0)

    wx = w_ref[:, 0:1]
    wy = w_ref[:, 1:2]
    wz = w_ref[:, 2:3]
    px = p_ref[0:1, :]
    py = p_ref[1:2, :]
    pz = p_ref[2:3, :]

    d = (jnp.abs(wx - px) + jnp.abs(wy - py)) + jnp.abs(wz - pz)

    rmin = jnp.min(d, axis=1, keepdims=True)
    jidx = jax.lax.broadcasted_iota(jnp.int32, d.shape, 1)
    ridx = jnp.min(jnp.where(d == rmin, jidx, _N), axis=1, keepdims=True)
    cham_ref[...] = rmin
    idx_ref[...] = ridx

    cmin = jnp.min(d, axis=0, keepdims=True)

    @pl.when(step == 0)
    def _init():
        colmin_ref[...] = cmin
        rowsum_ref[0, 0] = jnp.sum(rmin)

    @pl.when(step != 0)
    def _acc():
        colmin_ref[...] = jnp.minimum(colmin_ref[...], cmin)
        rowsum_ref[0, 0] = rowsum_ref[0, 0] + jnp.sum(rmin)

    @pl.when(step == nsteps - 1)
    def _final():
        mean_x = rowsum_ref[0, 0] / _N
        mean_y = jnp.sum(colmin_ref[...]) / _N
        loss_ref[...] = jnp.full((1, 1), (mean_x + mean_y) * 0.5, jnp.float32)


@functools.partial(jax.jit, static_argnums=())
def _chamfer(warped, p2rows):
    grid = _N // _TI
    cham, idx, _, loss = pl.pallas_call(
        _chamfer_body,
        grid=(grid,),
        in_specs=[
            pl.BlockSpec((_TI, 3), lambda i: (i, 0)),
            pl.BlockSpec((8, _N), lambda i: (0, 0)),
        ],
        out_specs=[
            pl.BlockSpec((_TI, 1), lambda i: (i, 0)),
            pl.BlockSpec((_TI, 1), lambda i: (i, 0)),
            pl.BlockSpec((1, _N), lambda i: (0, 0)),
            pl.BlockSpec((1, 1), lambda i: (0, 0)),
        ],
        out_shape=[
            jax.ShapeDtypeStruct((_N, 1), jnp.float32),
            jax.ShapeDtypeStruct((_N, 1), jnp.int32),
            jax.ShapeDtypeStruct((1, _N), jnp.float32),
            jax.ShapeDtypeStruct((1, 1), jnp.float32),
        ],
        scratch_shapes=[pltpu.SMEM((1, 1), jnp.float32)],
    )(warped, p2rows)
    return cham, idx, loss


def kernel(pc1, pc2, est_flow):
    warped = (pc1 + est_flow).reshape(_N, 3)
    p2rows = jnp.zeros((8, _N), jnp.float32).at[0:3, :].set(
        pc2.reshape(_N, 3).T)
    cham, idx, loss = _chamfer(warped, p2rows)
    return (loss[0, 0], cham.reshape(1, _N), idx.reshape(1, _N))
